# baseline (device time: 234148 ns/iter reference)
import jax
import jax.numpy as jnp
from jax import lax
from jax.experimental import pallas as pl
from jax.experimental.pallas import tpu as pltpu


K = 16


def kernel(x):
    _, m, n = x.shape
    n_out = n // 2
    m_half = m // 2
    c = m_half // K

    def body(x_ref, dummy_ref, out_ref, recv_buf, local_buf, y_recv_buf,
             x_send_sem, x_recv_sems, y_send_sem, y_recv_sems,
             in_sem, out_sem):
        my_x = lax.axis_index("x")
        my_y = lax.axis_index("y")
        row0 = my_y * m_half
        peer_row0 = (1 - my_y) * m_half
        my_col0 = my_x * n_out
        peer_col0 = (1 - my_x) * n_out

        barrier_sem = pltpu.get_barrier_semaphore()
        pl.semaphore_signal(barrier_sem, inc=1, device_id=(1 - my_x, my_y),
                            device_id_type=pl.DeviceIdType.MESH)
        pl.semaphore_signal(barrier_sem, inc=1, device_id=(my_x, 1 - my_y),
                            device_id_type=pl.DeviceIdType.MESH)
        pl.semaphore_wait(barrier_sem, 2)

        cp_in = pltpu.make_async_copy(
            x_ref.at[0, pl.ds(row0, m_half), pl.ds(my_col0, n_out)],
            local_buf, in_sem)
        cp_in.start()

        x_rdmas = []
        for i in range(K):
            r = pltpu.make_async_remote_copy(
                src_ref=x_ref.at[0, pl.ds(row0 + i * c, c),
                                 pl.ds(peer_col0, n_out)],
                dst_ref=recv_buf.at[pl.ds(i * c, c), :],
                send_sem=x_send_sem,
                recv_sem=x_recv_sems.at[i],
                device_id=(1 - my_x, my_y),
                device_id_type=pl.DeviceIdType.MESH,
            )
            r.start()
            x_rdmas.append(r)

        cp_in.wait()

        y_rdmas, out_cps = [], []
        for i in range(K):
            x_rdmas[i].wait_recv()
            recv_buf[pl.ds(i * c, c), :] = (
                recv_buf[pl.ds(i * c, c), :] + local_buf[pl.ds(i * c, c), :])
            ry = pltpu.make_async_remote_copy(
                src_ref=recv_buf.at[pl.ds(i * c, c), :],
                dst_ref=y_recv_buf.at[pl.ds(i * c, c), :],
                send_sem=y_send_sem,
                recv_sem=y_recv_sems.at[i],
                device_id=(my_x, 1 - my_y),
                device_id_type=pl.DeviceIdType.MESH,
            )
            ry.start()
            y_rdmas.append(ry)
            cp = pltpu.make_async_copy(
                recv_buf.at[pl.ds(i * c, c), :],
                out_ref.at[pl.ds(row0 + i * c, c), :], out_sem)
            cp.start()
            out_cps.append(cp)
            if i >= 1:
                y_rdmas[i - 1].wait_recv()
                cpp = pltpu.make_async_copy(
                    y_recv_buf.at[pl.ds((i - 1) * c, c), :],
                    out_ref.at[pl.ds(peer_row0 + (i - 1) * c, c), :], out_sem)
                cpp.start()
                out_cps.append(cpp)

        y_rdmas[K - 1].wait_recv()
        cpp = pltpu.make_async_copy(
            y_recv_buf.at[pl.ds((K - 1) * c, c), :],
            out_ref.at[pl.ds(peer_row0 + (K - 1) * c, c), :], out_sem)
        cpp.start()
        out_cps.append(cpp)

        for i in range(K):
            x_rdmas[i].wait_send()
            y_rdmas[i].wait_send()
        for cp in out_cps:
            cp.wait()

    dummy = jnp.zeros((m, n_out), x.dtype)
    return pl.pallas_call(
        body,
        out_shape=jax.ShapeDtypeStruct((m, n_out), x.dtype),
        in_specs=[pl.BlockSpec(memory_space=pl.ANY),
                  pl.BlockSpec(memory_space=pl.ANY)],
        out_specs=pl.BlockSpec(memory_space=pl.ANY),
        input_output_aliases={1: 0},
        scratch_shapes=[
            pltpu.VMEM((m_half, n_out), x.dtype),
            pltpu.VMEM((m_half, n_out), x.dtype),
            pltpu.VMEM((m_half, n_out), x.dtype),
            pltpu.SemaphoreType.DMA,
            pltpu.SemaphoreType.DMA((K,)),
            pltpu.SemaphoreType.DMA,
            pltpu.SemaphoreType.DMA((K,)),
            pltpu.SemaphoreType.DMA,
            pltpu.SemaphoreType.DMA,
        ],
        compiler_params=pltpu.CompilerParams(
            collective_id=0,
            vmem_limit_bytes=56 * 1024 * 1024,
        ),
    )(x, dummy)


# device time: 215973 ns/iter; 1.0842x vs baseline; 1.0842x over previous
import jax
import jax.numpy as jnp
from jax import lax
from jax.experimental import pallas as pl
from jax.experimental.pallas import tpu as pltpu

K = 32


def kernel(x):
    _, m, n = x.shape
    n_out = n // 2
    m_half = m // 2
    c = m_half // K

    def body(x_ref, out_ref, recv_buf, local_buf,
             x_send_sem, x_recv_sems, y_send_sem, y_recv_sem,
             in_sem, out_sem):
        my_x = lax.axis_index("x")
        my_y = lax.axis_index("y")
        row0 = my_y * m_half
        my_col0 = my_x * n_out
        peer_col0 = (1 - my_x) * n_out

        barrier_sem = pltpu.get_barrier_semaphore()
        pl.semaphore_signal(barrier_sem, inc=1, device_id=(1 - my_x, my_y),
                            device_id_type=pl.DeviceIdType.MESH)
        pl.semaphore_signal(barrier_sem, inc=1, device_id=(my_x, 1 - my_y),
                            device_id_type=pl.DeviceIdType.MESH)
        pl.semaphore_wait(barrier_sem, 2)

        cp_in = pltpu.make_async_copy(
            x_ref.at[0, pl.ds(row0, m_half), pl.ds(my_col0, n_out)],
            local_buf, in_sem)
        cp_in.start()

        x_rdmas = []
        for i in range(K):
            r = pltpu.make_async_remote_copy(
                src_ref=x_ref.at[0, pl.ds(row0 + i * c, c),
                                 pl.ds(peer_col0, n_out)],
                dst_ref=recv_buf.at[pl.ds(i * c, c), :],
                send_sem=x_send_sem,
                recv_sem=x_recv_sems.at[i],
                device_id=(1 - my_x, my_y),
                device_id_type=pl.DeviceIdType.MESH,
            )
            r.start()
            x_rdmas.append(r)

        cp_in.wait()

        y_rdmas, out_cps = [], []
        for i in range(K):
            x_rdmas[i].wait_recv()
            recv_buf[pl.ds(i * c, c), :] = (
                recv_buf[pl.ds(i * c, c), :] + local_buf[pl.ds(i * c, c), :])
            ry = pltpu.make_async_remote_copy(
                src_ref=recv_buf.at[pl.ds(i * c, c), :],
                dst_ref=out_ref.at[pl.ds(row0 + i * c, c), :],
                send_sem=y_send_sem,
                recv_sem=y_recv_sem,
                device_id=(my_x, 1 - my_y),
                device_id_type=pl.DeviceIdType.MESH,
            )
            ry.start()
            y_rdmas.append(ry)
            cp = pltpu.make_async_copy(
                recv_buf.at[pl.ds(i * c, c), :],
                out_ref.at[pl.ds(row0 + i * c, c), :], out_sem)
            cp.start()
            out_cps.append(cp)

        for i in range(K):
            x_rdmas[i].wait_send()
            y_rdmas[i].wait()
            out_cps[i].wait()

    return pl.pallas_call(
        body,
        out_shape=jax.ShapeDtypeStruct((m, n_out), x.dtype),
        in_specs=[pl.BlockSpec(memory_space=pl.ANY)],
        out_specs=pl.BlockSpec(memory_space=pl.ANY),
        scratch_shapes=[
            pltpu.VMEM((m_half, n_out), x.dtype),
            pltpu.VMEM((m_half, n_out), x.dtype),
            pltpu.SemaphoreType.DMA,
            pltpu.SemaphoreType.DMA((K,)),
            pltpu.SemaphoreType.DMA,
            pltpu.SemaphoreType.DMA,
            pltpu.SemaphoreType.DMA,
            pltpu.SemaphoreType.DMA,
        ],
        compiler_params=pltpu.CompilerParams(
            collective_id=0,
            vmem_limit_bytes=40 * 1024 * 1024,
        ),
    )(x)
